# baseline (speedup 1.0000x reference)
import jax
import jax.numpy as jnp
from jax import lax
from jax.experimental import pallas as pl
from jax.experimental.pallas import tpu as pltpu
from jax.experimental.pallas import tpu_sc as plsc

B = 16
L = 256
T = B * L
H = 16
G = 4
D = 128
NREP = H // G
NUM_BLOCKS = 128
BS = 256
SCALE = float(1.0 / (D ** 0.5))
NEG = -100.0


def _attn_body(q_ref, k_ref, v_ref, o_ref):
    rows = lax.broadcasted_iota(jnp.int32, (L, L), 0)
    cols = lax.broadcasted_iota(jnp.int32, (L, L), 1)
    causal = rows >= cols
    for g in range(G):
        k2 = k_ref[:, g, :]
        v2 = v_ref[:, g, :]
        for r in range(NREP):
            h = g * NREP + r
            qj = q_ref[:, h, :]
            s = lax.dot_general(qj, k2, (((1,), (1,)), ((), ())),
                                preferred_element_type=jnp.float32) * SCALE
            s = jnp.clip(s, -100.0, 100.0)
            s = jnp.where(causal, s, NEG)
            m = jnp.max(s, axis=1, keepdims=True)
            p = jnp.exp(s - m)
            denom = jnp.sum(p, axis=1, keepdims=True)
            o = lax.dot_general(p, v2, (((1,), (0,)), ((), ())),
                                preferred_element_type=jnp.float32)
            o_ref[:, pl.ds(h * D, D)] = o / denom


_attention = pl.pallas_call(
    _attn_body,
    grid=(B,),
    in_specs=[
        pl.BlockSpec((L, H, D), lambda b: (b, 0, 0)),
        pl.BlockSpec((L, G, D), lambda b: (b, 0, 0)),
        pl.BlockSpec((L, G, D), lambda b: (b, 0, 0)),
    ],
    out_specs=pl.BlockSpec((L, H * D), lambda b: (b, 0)),
    out_shape=jax.ShapeDtypeStruct((T, H * D), jnp.float32),
    compiler_params=pltpu.CompilerParams(
        dimension_semantics=("parallel",)),
)


TILES = 16
NW = 2 * TILES
BLK_PER_W = NUM_BLOCKS // NW
TOK_PER_TILE = T // TILES
CC = 32
CPB = BS // CC
NCOPY = BLK_PER_W * CPB
NSCAT = TOK_PER_TILE // CC


def _row_dmas(op, buf, idx, sem, cid, cout=None, data=None, tok0=None,
              ib=0):
    def group(g, carry):
        vec = idx[pl.ds(ib + g * 16, 16)]
        for lane in range(16):
            j = g * 16 + lane
            s = vec[lane]
            bi = lax.shift_right_logical(s, 8)
            ii = lax.bitwise_and(s, BS - 1)

            @pl.when(lax.shift_right_logical(bi, 6) == cid)
            def _():
                if cout is not None:
                    op(pltpu.make_async_copy(buf.at[j], cout.at[bi, ii], sem))
                else:
                    op(pltpu.make_async_copy(data.at[tok0 + j], buf.at[j],
                                             sem))
        return carry

    lax.fori_loop(0, CC // 16, group, 0)


class _RowDmaHandle:
    def __init__(self, *args, **kw):
        self.args, self.kw = args, kw
        _row_dmas(lambda d: d.start(), *args, **kw)

    def wait(self):
        _row_dmas(lambda d: d.wait(), *self.args, **self.kw)


def _stream(chunks, bufs, sin, sout):
    nchunks = len(chunks)
    nb = len(bufs)
    pre = min(nb - 1, nchunks)
    d_in = [None] * nchunks
    d_out = [None] * nchunks
    waited = [False] * nchunks
    for c in range(pre):
        d_in[c] = chunks[c][0](bufs[c % nb], sin[c % nb])
    for c in range(nchunks):
        n = c + pre
        if n < nchunks:
            if n - nb >= 0:
                d_out[n - nb].wait()
                waited[n - nb] = True
            d_in[n] = chunks[n][0](bufs[n % nb], sin[n % nb])
        d_in[c].wait()
        d_out[c] = chunks[c][1](bufs[c % nb], sout[c % nb])
    for c in range(nchunks):
        if not waited[c]:
            d_out[c].wait()


def _sc_body(k3, v3, slots, kc_in, vc_in, kc_out, vc_out,
             buf_a, buf_b, buf_c, buf_d, idx_all,
             sin_a, sin_b, sin_c, sin_d, sout_a, sout_b, sout_c, sout_d):
    cid = lax.axis_index("c")
    sid = lax.axis_index("s")
    w = cid * TILES + sid
    bufs = (buf_a, buf_b, buf_c, buf_d)
    sin = (sin_a, sin_b, sin_c, sin_d)
    sout = (sout_a, sout_b, sout_c, sout_d)
    blk0 = w * BLK_PER_W
    tb = sid * TOK_PER_TILE

    def copy_chunk(cin, cout, c):
        sl = lambda cache: cache.at[blk0 + c // CPB, pl.ds((c % CPB) * CC, CC)]
        return (lambda buf, sem: pltpu.async_copy(sl(cin), buf, sem),
                lambda buf, sem: pltpu.async_copy(buf, sl(cout), sem))

    def scat_chunk(data, cout, c):
        return (lambda buf, sem: _RowDmaHandle(
                    buf, idx_all, sem, cid, data=data, tok0=tb + c * CC,
                    ib=c * CC),
                lambda buf, sem: _RowDmaHandle(
                    buf, idx_all, sem, cid, cout=cout, ib=c * CC))

    pltpu.sync_copy(slots.at[pl.ds(tb, TOK_PER_TILE)], idx_all)

    _stream([copy_chunk(kc_in, kc_out, c) for c in range(NCOPY)],
            bufs, sin, sout)

    plsc.subcore_barrier()

    chunks = []
    for c in range(NCOPY):
        chunks.append(copy_chunk(vc_in, vc_out, c))
        if c % (NCOPY // NSCAT) == NCOPY // NSCAT - 1:
            chunks.append(scat_chunk(k3, kc_out, c // (NCOPY // NSCAT)))
    _stream(chunks, bufs, sin, sout)

    plsc.subcore_barrier()

    _stream([scat_chunk(v3, vc_out, c) for c in range(NSCAT)],
            bufs, sin, sout)


_scatter = pl.kernel(
    _sc_body,
    out_type=(jax.ShapeDtypeStruct((NUM_BLOCKS, BS, G, D), jnp.float32),
              jax.ShapeDtypeStruct((NUM_BLOCKS, BS, G, D), jnp.float32)),
    mesh=plsc.VectorSubcoreMesh(core_axis_name="c", subcore_axis_name="s"),
    scratch_types=[
        pltpu.VMEM((CC, G, D), jnp.float32),
        pltpu.VMEM((CC, G, D), jnp.float32),
        pltpu.VMEM((CC, G, D), jnp.float32),
        pltpu.VMEM((CC, G, D), jnp.float32),
        pltpu.VMEM((TOK_PER_TILE,), jnp.int32),
        pltpu.SemaphoreType.DMA,
        pltpu.SemaphoreType.DMA,
        pltpu.SemaphoreType.DMA,
        pltpu.SemaphoreType.DMA,
        pltpu.SemaphoreType.DMA,
        pltpu.SemaphoreType.DMA,
        pltpu.SemaphoreType.DMA,
        pltpu.SemaphoreType.DMA,
    ],
)


def kernel(q, k, v, k_cache, v_cache, slot_mapping, cu_seqlens_q, cu_seqlens_k):
    kc, vc = _scatter(k, v, slot_mapping, k_cache, v_cache)
    o = _attention(q, k, v)
    return o, kc, vc


# speedup vs baseline: 1.0219x; 1.0219x over previous
import jax
import jax.numpy as jnp
from jax import lax
from jax.experimental import pallas as pl
from jax.experimental.pallas import tpu as pltpu
from jax.experimental.pallas import tpu_sc as plsc

B = 16
L = 256
T = B * L
H = 16
G = 4
D = 128
NREP = H // G
NUM_BLOCKS = 128
BS = 256
SCALE = float(1.0 / (D ** 0.5))
NEG = -100.0


def _attn_body(q_ref, k_ref, v_ref, o_ref):
    rows = lax.broadcasted_iota(jnp.int32, (L, L), 0)
    cols = lax.broadcasted_iota(jnp.int32, (L, L), 1)
    causal = rows >= cols
    for g in range(G):
        k2 = k_ref[:, g, :]
        v2 = v_ref[:, g, :]
        for r in range(NREP):
            h = g * NREP + r
            qj = q_ref[:, h, :]
            s = lax.dot_general(qj, k2, (((1,), (1,)), ((), ())),
                                preferred_element_type=jnp.float32) * SCALE
            s = jnp.clip(s, -100.0, 100.0)
            s = jnp.where(causal, s, NEG)
            m = jnp.max(s, axis=1, keepdims=True)
            p = jnp.exp(s - m)
            denom = jnp.sum(p, axis=1, keepdims=True)
            o = lax.dot_general(p, v2, (((1,), (0,)), ((), ())),
                                preferred_element_type=jnp.float32)
            o_ref[:, pl.ds(h * D, D)] = o / denom


_attention = pl.pallas_call(
    _attn_body,
    grid=(B,),
    in_specs=[
        pl.BlockSpec((L, H, D), lambda b: (b, 0, 0)),
        pl.BlockSpec((L, G, D), lambda b: (b, 0, 0)),
        pl.BlockSpec((L, G, D), lambda b: (b, 0, 0)),
    ],
    out_specs=pl.BlockSpec((L, H * D), lambda b: (b, 0)),
    out_shape=jax.ShapeDtypeStruct((T, H * D), jnp.float32),
    compiler_params=pltpu.CompilerParams(
        dimension_semantics=("parallel",)),
)


TILES = 16
NW = 2 * TILES
BLK_PER_W = NUM_BLOCKS // NW
TOK_PER_TILE = T // TILES
CC = 64
CPB = BS // CC
NCOPY = BLK_PER_W * CPB
NSCAT = TOK_PER_TILE // CC


def _row_dmas(op, buf, idx, sem, cid, cout=None, data=None, tok0=None):
    def group(g, carry):
        vec = idx[pl.ds(g * 16, 16)]
        for lane in range(16):
            j = g * 16 + lane
            s = vec[lane]
            bi = lax.shift_right_logical(s, 8)
            ii = lax.bitwise_and(s, BS - 1)

            @pl.when(lax.shift_right_logical(bi, 6) == cid)
            def _():
                if cout is not None:
                    op(pltpu.make_async_copy(buf.at[j], cout.at[bi, ii], sem))
                else:
                    op(pltpu.make_async_copy(data.at[tok0 + j], buf.at[j],
                                             sem))
        return carry

    lax.fori_loop(0, CC // 16, group, 0)


class _RowDmaHandle:
    def __init__(self, *args, **kw):
        self.args, self.kw = args, kw
        _row_dmas(lambda d: d.start(), *args, **kw)

    def wait(self):
        _row_dmas(lambda d: d.wait(), *self.args, **self.kw)


def _stream(chunks, bufs, sin, sout):
    nchunks = len(chunks)
    nb = len(bufs)
    pre = min(nb - 1, nchunks)
    d_in = [None] * nchunks
    d_out = [None] * nchunks
    waited = [False] * nchunks
    for c in range(pre):
        d_in[c] = chunks[c][0](bufs[c % nb], sin[c % nb])
    for c in range(nchunks):
        n = c + pre
        if n < nchunks:
            if n - nb >= 0:
                d_out[n - nb].wait()
                waited[n - nb] = True
            d_in[n] = chunks[n][0](bufs[n % nb], sin[n % nb])
        d_in[c].wait()
        d_out[c] = chunks[c][1](bufs[c % nb], sout[c % nb])
    for c in range(nchunks):
        if not waited[c]:
            d_out[c].wait()


def _sc_body(k3, v3, slots, kc_in, vc_in, kc_out, vc_out,
             buf_a, buf_b, buf_c, idx0, idx1, idx2, idx3,
             sin_a, sin_b, sin_c, sout_a, sout_b, sout_c):
    cid = lax.axis_index("c")
    sid = lax.axis_index("s")
    w = cid * TILES + sid
    bufs = (buf_a, buf_b, buf_c)
    sin = (sin_a, sin_b, sin_c)
    sout = (sout_a, sout_b, sout_c)
    blk0 = w * BLK_PER_W
    tb = sid * TOK_PER_TILE
    idxs = (idx0, idx1, idx2, idx3)

    def copy_chunk(cin, cout, c):
        sl = lambda cache: cache.at[blk0 + c // CPB, pl.ds((c % CPB) * CC, CC)]
        return (lambda buf, sem: pltpu.async_copy(sl(cin), buf, sem),
                lambda buf, sem: pltpu.async_copy(buf, sl(cout), sem))

    def scat_chunk(data, cout, c):
        return (lambda buf, sem: _RowDmaHandle(
                    buf, idxs[c], sem, cid, data=data, tok0=tb + c * CC),
                lambda buf, sem: _RowDmaHandle(
                    buf, idxs[c], sem, cid, cout=cout))

    for j in range(NSCAT):
        pltpu.sync_copy(slots.at[pl.ds(tb + j * CC, CC)], idxs[j])

    _stream([copy_chunk(kc_in, kc_out, c) for c in range(NCOPY)],
            bufs, sin, sout)

    plsc.subcore_barrier()

    chunks = []
    for c in range(NCOPY):
        chunks.append(copy_chunk(vc_in, vc_out, c))
        if c % CPB == CPB - 1:
            chunks.append(scat_chunk(k3, kc_out, c // CPB))
    _stream(chunks, bufs, sin, sout)

    plsc.subcore_barrier()

    _stream([scat_chunk(v3, vc_out, c) for c in range(NSCAT)],
            bufs, sin, sout)


_scatter = pl.kernel(
    _sc_body,
    out_type=(jax.ShapeDtypeStruct((NUM_BLOCKS, BS, G, D), jnp.float32),
              jax.ShapeDtypeStruct((NUM_BLOCKS, BS, G, D), jnp.float32)),
    mesh=plsc.VectorSubcoreMesh(core_axis_name="c", subcore_axis_name="s"),
    scratch_types=[
        pltpu.VMEM((CC, G, D), jnp.float32),
        pltpu.VMEM((CC, G, D), jnp.float32),
        pltpu.VMEM((CC, G, D), jnp.float32),
        pltpu.VMEM((CC,), jnp.int32),
        pltpu.VMEM((CC,), jnp.int32),
        pltpu.VMEM((CC,), jnp.int32),
        pltpu.VMEM((CC,), jnp.int32),
        pltpu.SemaphoreType.DMA,
        pltpu.SemaphoreType.DMA,
        pltpu.SemaphoreType.DMA,
        pltpu.SemaphoreType.DMA,
        pltpu.SemaphoreType.DMA,
        pltpu.SemaphoreType.DMA,
    ],
)


def kernel(q, k, v, k_cache, v_cache, slot_mapping, cu_seqlens_q, cu_seqlens_k):
    kc, vc = _scatter(k, v, slot_mapping, k_cache, v_cache)
    o = _attention(q, k, v)
    return o, kc, vc
